# Initial kernel scaffold; baseline (speedup 1.0000x reference)
import jax
import jax.numpy as jnp
from jax.experimental import pallas as pl


def kernel(inputs, edge_index, W_self1, W_neigh1, b1, W_self2, W_neigh2, b2):
    raise NotImplementedError("write your pallas kernel here")


# speedup vs baseline: 6.5568x; 6.5568x over previous
import functools

import jax
import jax.numpy as jnp
from jax import lax
from jax.experimental import pallas as pl
from jax.experimental.pallas import tpu as pltpu
from jax.experimental.pallas import tpu_sc as plsc

N_NODES = 10000
N_EDGES = 320000
D = 128

NS = 16
CHUNK = 128
CHUNKS_PW = 158
E_PAD = NS * CHUNKS_PW * CHUNK
N_PAD = 10112
ROWS_PER_SUB = N_PAD // NS
DHALF1 = 80
DHALF2 = 64


def _make_sc_agg(d):
  mesh = plsc.VectorSubcoreMesh(core_axis_name="c", subcore_axis_name="s")

  @functools.partial(
      pl.kernel,
      mesh=mesh,
      out_type=jax.ShapeDtypeStruct((2, N_PAD, d), jnp.float32),
      scratch_types=[
          pltpu.VMEM((CHUNKS_PW, CHUNK), jnp.int32),
          pltpu.VMEM((CHUNKS_PW, CHUNK), jnp.int32),
          pltpu.VMEM((CHUNK, d), jnp.float32),
          pltpu.VMEM((CHUNK, d), jnp.float32),
          pltpu.VMEM_SHARED((N_PAD, d), jnp.float32),
          pltpu.SemaphoreType.DMA,
          pltpu.SemaphoreType.DMA,
      ],
      compiler_params=pltpu.CompilerParams(use_tc_tiling_on_sc=False),
  )
  def sc_agg(zeros_hbm, tl_hbm, tr_hbm, src_hbm, dst_hbm, out_hbm,
             srcv, dstv, b0, b1, acc, s0, s1):
    cid = lax.axis_index("c")
    sid = lax.axis_index("s")

    pltpu.sync_copy(src_hbm.at[sid], srcv)
    pltpu.sync_copy(dst_hbm.at[sid], dstv)

    row0 = sid * ROWS_PER_SUB
    pltpu.sync_copy(zeros_hbm.at[pl.ds(row0, ROWS_PER_SUB)],
                    acc.at[pl.ds(row0, ROWS_PER_SUB)])
    plsc.subcore_barrier()

    def start_gather(g, buf, sem):
      @pl.when(cid == 0)
      def _():
        pltpu.async_copy(tl_hbm.at[srcv.at[g]], buf, sem)

      @pl.when(cid == 1)
      def _():
        pltpu.async_copy(tr_hbm.at[srcv.at[g]], buf, sem)

    def wait_gather(buf, sem):
      pltpu.make_async_copy(tl_hbm.at[srcv.at[0]], buf, sem).wait()

    start_gather(0, b0, s0)

    def body(t, _):
      g0 = 2 * t
      start_gather(g0 + 1, b1, s1)
      wait_gather(b0, s0)
      pltpu.sync_copy(b0, acc.at[dstv.at[g0]], add=True)

      @pl.when(t + 1 < CHUNKS_PW // 2)
      def _():
        start_gather(g0 + 2, b0, s0)

      wait_gather(b1, s1)
      pltpu.sync_copy(b1, acc.at[dstv.at[g0 + 1]], add=True)
      return 0

    lax.fori_loop(0, CHUNKS_PW // 2, body, 0)
    plsc.subcore_barrier()

    pltpu.sync_copy(acc.at[pl.ds(row0, ROWS_PER_SUB)],
                    out_hbm.at[cid, pl.ds(row0, ROWS_PER_SUB)])

  return sc_agg


_sc_agg_l1 = _make_sc_agg(DHALF1)
_sc_agg_l2 = _make_sc_agg(DHALF2)

R = 1000


def _tc_layer1(x_ref, p_ref, ws_ref, wn_ref, b_ref, hl_ref, hr_ref, r_ref):
  p = p_ref[...]
  agg = jnp.concatenate([p[0], p[1][:, :D - DHALF1]], axis=1)
  deg = p[1][:, D - DHALF1:D - DHALF1 + 1]
  r = 1.0 / jnp.maximum(deg, 1.0)
  h = (jnp.dot(x_ref[...], ws_ref[...], preferred_element_type=jnp.float32)
       + jnp.dot(agg * r, wn_ref[...], preferred_element_type=jnp.float32)
       + b_ref[...])
  h = jnp.maximum(h, 0.0)
  hl_ref[...] = h[:, :DHALF2]
  hr_ref[...] = h[:, DHALF2:]
  r_ref[...] = jnp.broadcast_to(r, (R, 8))


def _tc_layer2(hl_ref, hr_ref, p_ref, r_ref, ws_ref, wn_ref, b_ref, o_ref):
  p = p_ref[...]
  mean = jnp.concatenate([p[0], p[1]], axis=1) * r_ref[:, :1]
  ws = ws_ref[...]
  o_ref[...] = (
      jnp.dot(hl_ref[...], ws[:DHALF2], preferred_element_type=jnp.float32)
      + jnp.dot(hr_ref[...], ws[DHALF2:], preferred_element_type=jnp.float32)
      + jnp.dot(mean, wn_ref[...], preferred_element_type=jnp.float32)
      + b_ref[...])


def _row_block(shape_tail):
  return pl.BlockSpec((R,) + shape_tail, lambda i: (i,) + (0,) * len(shape_tail))


def _part_block(d):
  return pl.BlockSpec((2, R, d), lambda i: (0, i, 0))


def _full_block(shape):
  return pl.BlockSpec(shape, lambda i: (0,) * len(shape))


def kernel(inputs, edge_index, W_self1, W_neigh1, b1, W_self2, W_neigh2, b2):
  x = inputs
  src = edge_index[0].astype(jnp.int32)
  dst = edge_index[1].astype(jnp.int32)
  pad = E_PAD - N_EDGES
  src = jnp.concatenate([src, jnp.zeros((pad,), jnp.int32)])
  dst = jnp.concatenate([dst, jnp.full((pad,), N_NODES, jnp.int32)])
  src3 = src.reshape(NS, CHUNKS_PW, CHUNK)
  dst3 = dst.reshape(NS, CHUNKS_PW, CHUNK)

  xl = x[:, :DHALF1]
  xr = jnp.concatenate(
      [x[:, DHALF1:], jnp.ones((N_NODES, 1), jnp.float32),
       jnp.zeros((N_NODES, 2 * DHALF1 - D - 1), jnp.float32)], axis=1)
  z1 = jnp.zeros((N_PAD, DHALF1), jnp.float32)
  p1 = _sc_agg_l1(z1, xl, xr, src3, dst3)

  hl, hr, rdeg = pl.pallas_call(
      _tc_layer1,
      grid=(N_NODES // R,),
      in_specs=[
          _row_block((D,)),
          _part_block(DHALF1),
          _full_block((D, D)),
          _full_block((D, D)),
          _full_block((1, D)),
      ],
      out_specs=[_row_block((DHALF2,)), _row_block((DHALF2,)),
                 _row_block((8,))],
      out_shape=[
          jax.ShapeDtypeStruct((N_NODES, DHALF2), jnp.float32),
          jax.ShapeDtypeStruct((N_NODES, DHALF2), jnp.float32),
          jax.ShapeDtypeStruct((N_NODES, 8), jnp.float32),
      ],
  )(x, p1, W_self1, W_neigh1, b1.reshape(1, D))

  z2 = jnp.zeros((N_PAD, DHALF2), jnp.float32)
  p2 = _sc_agg_l2(z2, hl, hr, src3, dst3)

  out = pl.pallas_call(
      _tc_layer2,
      grid=(N_NODES // R,),
      in_specs=[
          _row_block((DHALF2,)),
          _row_block((DHALF2,)),
          _part_block(DHALF2),
          _row_block((8,)),
          _full_block((D, D)),
          _full_block((D, D)),
          _full_block((1, D)),
      ],
      out_specs=_row_block((D,)),
      out_shape=jax.ShapeDtypeStruct((N_NODES, D), jnp.float32),
  )(hl, hr, p2, rdeg, W_self2, W_neigh2, b2.reshape(1, D))

  return out
